# Initial kernel scaffold; baseline (speedup 1.0000x reference)
import jax
import jax.numpy as jnp
from jax.experimental import pallas as pl


def kernel(input, weights):
    raise NotImplementedError("write your pallas kernel here")


# speedup vs baseline: 1.8437x; 1.8437x over previous
import functools

import jax
import jax.numpy as jnp
from jax import lax
from jax.experimental import pallas as pl
from jax.experimental.pallas import tpu as pltpu
from jax.experimental.pallas import tpu_sc as plsc

PAD = 1
BSZ = 4
SEQ = 4096
D = 1024
G = BSZ * SEQ
NW = 32
PER_W = G // NW
CHUNK = 64
NCHUNK = PER_W // CHUNK
L = 16


def _make_kernel():
    mesh = plsc.VectorSubcoreMesh(core_axis_name="c", subcore_axis_name="s")

    @functools.partial(
        pl.kernel,
        mesh=mesh,
        out_type=jax.ShapeDtypeStruct((G, D), jnp.float32),
        scratch_types=[
            pltpu.VMEM((PER_W,), jnp.int32),
            pltpu.VMEM((CHUNK,), jnp.int32),
            pltpu.VMEM((CHUNK, D), jnp.float32),
            pltpu.SemaphoreType.DMA,
        ],
    )
    def k(inp_hbm, weights_hbm, out_hbm, tok_v, idx_v, rows_v, sem):
        wid = lax.axis_index("s") * 2 + lax.axis_index("c")
        base = wid * PER_W
        joff = lax.rem(base, SEQ)

        pltpu.sync_copy(inp_hbm.at[pl.ds(base, PER_W)], tok_v)

        for c in range(NCHUNK):
            for v in range(CHUNK // L):
                tok = tok_v[pl.ds(c * CHUNK + v * L, L)]
                j = joff + c * CHUNK + v * L + lax.iota(jnp.int32, L)
                idx = jnp.where(tok != PAD, j + (PAD + 1), PAD)
                idx_v[pl.ds(v * L, L)] = idx
            pltpu.async_copy(weights_hbm.at[idx_v], rows_v, sem).wait()
            pltpu.sync_copy(rows_v, out_hbm.at[pl.ds(base + c * CHUNK, CHUNK)])

    return k


_gather = _make_kernel()


@jax.jit
def kernel(input, weights):
    out = _gather(input.reshape(-1), weights)
    return out.reshape(BSZ, SEQ, D)
